# baseline (speedup 1.0000x reference)
import functools

import jax
import jax.numpy as jnp
from jax import lax
from jax.experimental import pallas as pl
from jax.experimental.pallas import tpu as pltpu
from jax.experimental.pallas import tpu_sc as plsc

D_MODEL = 128
MAX_LEN = 32

_NC = 2
_NS = 16
_NW = _NC * _NS
_LANES = 16

_B = 4 * 2048 * 32
_BPW = _B // _NW
_GB = 128
_G = _BPW // _GB
_NBUF = 2
_VROWS = 2 * MAX_LEN + 1
_GELEM = _GB * D_MODEL


@functools.partial(
    pl.kernel,
    mesh=plsc.VectorSubcoreMesh(core_axis_name="c", subcore_axis_name="s"),
    compiler_params=pltpu.CompilerParams(needs_layout_passes=False),
    out_type=jax.ShapeDtypeStruct((_B * D_MODEL,), jnp.float32),
    scratch_types=[
        pltpu.VMEM((_G, _GB), jnp.int32),
        pltpu.VMEM((_GELEM,), jnp.float32),
        pltpu.VMEM((_GELEM,), jnp.float32),
        pltpu.VMEM((_VROWS * D_MODEL,), jnp.float32),
        pltpu.SemaphoreType.DMA,
        pltpu.SemaphoreType.DMA,
        pltpu.SemaphoreType.DMA,
    ],
)
def _rpe_lookup(off_hbm, emb_hbm, out_hbm, idx_v, stage0, stage1, table_v, sem_in, *so):
    stages = (stage0, stage1)
    wid = lax.axis_index("s") * _NC + lax.axis_index("c")

    pltpu.async_copy(emb_hbm, table_v, sem_in)
    pltpu.sync_copy(off_hbm.at[wid], idx_v)
    pltpu.make_async_copy(emb_hbm, table_v, sem_in).wait()

    base = wid * _BPW * D_MODEL
    lane_rows = lax.iota(jnp.int32, _LANES) * D_MODEL

    def fire_o(g, j):
        pltpu.async_copy(
            stages[j], out_hbm.at[pl.ds(base + g * _GELEM, _GELEM)], so[j]
        )

    def wait_o(g, j):
        pltpu.make_async_copy(
            stages[j], out_hbm.at[pl.ds(base + g * _GELEM, _GELEM)], so[j]
        ).wait()

    def fill(g, j):
        def block_body(b, carry):
            vec = idx_v[g, pl.ds(b * _LANES, _LANES)]
            t = jnp.minimum(jnp.maximum(vec + MAX_LEN, 0), 2 * MAX_LEN)
            src = t * D_MODEL
            dst = lane_rows + b * (_LANES * D_MODEL)
            for col in range(D_MODEL):
                v = plsc.load_gather(table_v, [src + col])
                plsc.store_scatter(stages[j], [dst + col], v)
            return carry

        lax.fori_loop(0, _GB // _LANES, block_body, 0)

    fill(0, 0)
    fire_o(0, 0)
    fill(1, 1)
    fire_o(1, 1)

    def main_body(p, carry):
        for u in range(_NBUF):
            g = _NBUF + _NBUF * p + u
            wait_o(g - _NBUF, u)
            fill(g, u)
            fire_o(g, u)
        return carry

    lax.fori_loop(0, (_G - _NBUF) // _NBUF, main_body, 0)

    for u in range(_NBUF):
        wait_o(_G - _NBUF + u, u)


def kernel(offset, embedding):
    off = offset.reshape(_NW, _G, _GB).astype(jnp.int32)
    out = _rpe_lookup(off, embedding.reshape(-1))
    return out.reshape(offset.shape + (D_MODEL,))


# speedup vs baseline: 2.8313x; 2.8313x over previous
import functools

import jax
import jax.numpy as jnp
from jax import lax
from jax.experimental import pallas as pl
from jax.experimental.pallas import tpu as pltpu
from jax.experimental.pallas import tpu_sc as plsc

D_MODEL = 128
MAX_LEN = 32

_NC = 2
_NS = 16
_NW = _NC * _NS
_LANES = 16

_B = 4 * 2048 * 32
_BPW = _B // _NW
_GB = 128
_G = _BPW // _GB
_NBUF = 2
_VROWS = 2 * MAX_LEN + 1
_PAD = 136
_GELEM = _GB * D_MODEL


@functools.partial(
    pl.kernel,
    mesh=plsc.VectorSubcoreMesh(core_axis_name="c", subcore_axis_name="s"),
    compiler_params=pltpu.CompilerParams(needs_layout_passes=False),
    out_type=jax.ShapeDtypeStruct((_B * D_MODEL,), jnp.float32),
    scratch_types=[
        pltpu.VMEM((_G, _GB), jnp.int32),
        pltpu.VMEM((_GELEM,), jnp.float32),
        pltpu.VMEM((_GELEM,), jnp.float32),
        pltpu.VMEM((_VROWS * _PAD,), jnp.float32),
        pltpu.SemaphoreType.DMA,
        pltpu.SemaphoreType.DMA,
        pltpu.SemaphoreType.DMA,
    ],
)
def _rpe_lookup(off_hbm, emb_hbm, out_hbm, idx_v, stage0, stage1, table_v, sem_in, *so):
    stages = (stage0, stage1)
    wid = lax.axis_index("s") * _NC + lax.axis_index("c")

    pltpu.async_copy(emb_hbm, table_v, sem_in)
    pltpu.sync_copy(off_hbm.at[wid], idx_v)
    pltpu.make_async_copy(emb_hbm, table_v, sem_in).wait()

    base = wid * _BPW * D_MODEL
    lane = lax.iota(jnp.int32, _LANES)
    lane_rows = lane * D_MODEL

    def fire_o(g, j):
        pltpu.async_copy(
            stages[j], out_hbm.at[pl.ds(base + g * _GELEM, _GELEM)], so[j]
        )

    def wait_o(g, j):
        pltpu.make_async_copy(
            stages[j], out_hbm.at[pl.ds(base + g * _GELEM, _GELEM)], so[j]
        ).wait()

    def fill(g, j):
        def block_body(b, carry):
            vec = idx_v[g, pl.ds(b * _LANES, _LANES)]
            t = jnp.minimum(jnp.maximum(vec + MAX_LEN, 0), 2 * MAX_LEN)
            src_row = t * _PAD
            dst_row = lane_rows + b * (_LANES * D_MODEL)
            for col in range(D_MODEL):
                colv = (col + lane) & (D_MODEL - 1)
                v = plsc.load_gather(table_v, [src_row + colv])
                plsc.store_scatter(stages[j], [dst_row + colv], v)
            return carry

        lax.fori_loop(0, _GB // _LANES, block_body, 0)

    fill(0, 0)
    fire_o(0, 0)
    fill(1, 1)
    fire_o(1, 1)

    def main_body(p, carry):
        for u in range(_NBUF):
            g = _NBUF + _NBUF * p + u
            wait_o(g - _NBUF, u)
            fill(g, u)
            fire_o(g, u)
        return carry

    lax.fori_loop(0, (_G - _NBUF) // _NBUF, main_body, 0)

    for u in range(_NBUF):
        wait_o(_G - _NBUF + u, u)


def kernel(offset, embedding):
    off = offset.reshape(_NW, _G, _GB).astype(jnp.int32)
    emb = jnp.pad(embedding, ((0, 0), (0, _PAD - D_MODEL))).reshape(-1)
    out = _rpe_lookup(off, emb)
    return out.reshape(offset.shape + (D_MODEL,))
